# Initial kernel scaffold; baseline (speedup 1.0000x reference)
import functools

import jax
import jax.numpy as jnp
from jax import lax
from jax.experimental import pallas as pl
from jax.experimental.pallas import tpu as pltpu
from jax.experimental.pallas import tpu_sc as plsc

_VOCAB = 100000
_DIM = 64
_NCLS = 100
_B = 4096
_L = 200

_NC = 2
_NS = 16
_NW = _NC * _NS
_BPW = _B // _NW


def _sc_pool_body(table_hbm, xw_hbm, out_hbm, idx_v, accum, sem):
    wid = lax.axis_index("s") * _NC + lax.axis_index("c")

    pltpu.sync_copy(xw_hbm.at[wid], idx_v)

    def zero_body(i, carry):
        for k in range(_DIM // 16):
            accum[i, pl.ds(k * 16, 16)] = jnp.zeros((16,), jnp.float32)
        return carry

    lax.fori_loop(0, _BPW, zero_body, 0)

    def fire(j, carry):
        pltpu.async_copy(table_hbm.at[idx_v.at[j]], accum, sem, add=True)
        return carry

    lax.fori_loop(0, _L, fire, 0)

    def drain(j, carry):
        pltpu.make_async_copy(table_hbm.at[idx_v.at[0]], accum, sem).wait()
        return carry

    lax.fori_loop(0, _L, drain, 0)

    pltpu.sync_copy(accum, out_hbm.at[pl.ds(wid * _BPW, _BPW)])


_sc_pool = functools.partial(
    pl.kernel,
    out_type=jax.ShapeDtypeStruct((_B, _DIM), jnp.float32),
    mesh=plsc.VectorSubcoreMesh(core_axis_name="c", subcore_axis_name="s"),
    scratch_types=[
        pltpu.VMEM((_L, _BPW), jnp.int32),
        pltpu.VMEM((_BPW, _DIM), jnp.float32),
        pltpu.SemaphoreType.DMA,
    ],
)(_sc_pool_body)


def _tc_head_body(ps_ref, x_ref, t0_ref, w_ref, b_ref, o_ref):
    cnt = jnp.sum((x_ref[...] == 0).astype(jnp.float32), axis=1, keepdims=True)
    pooled = ps_ref[...] - cnt * t0_ref[...]
    o_ref[...] = (1.0 / _L) * lax.dot_general(
        pooled, w_ref[...], (((1,), (1,)), ((), ())),
        preferred_element_type=jnp.float32,
    ) + b_ref[...]


_tc_head = pl.pallas_call(
    _tc_head_body,
    out_shape=jax.ShapeDtypeStruct((_B, _NCLS), jnp.float32),
)


def kernel(x, table, W, b):
    xi = x.astype(jnp.int32)
    xw = xi.T.reshape(_L, _NW, _BPW).transpose(1, 0, 2)
    pooled_sum = _sc_pool(table, xw)
    return _tc_head(pooled_sum, xi, table[0:1], W, b.reshape(1, _NCLS))


# speedup vs baseline: 17.1074x; 17.1074x over previous
import functools

import jax
import jax.numpy as jnp
from jax import lax
from jax.experimental import pallas as pl
from jax.experimental.pallas import tpu as pltpu
from jax.experimental.pallas import tpu_sc as plsc

_VOCAB = 100000
_DIM = 64
_NCLS = 100
_B = 4096
_L = 200

_NC = 2
_NS = 16
_NW = _NC * _NS
_BPW = _B // _NW


def _sc_pool_body(table_hbm, xw_hbm, out_hbm, idx_v, accum, sem):
    wid = lax.axis_index("s") * _NC + lax.axis_index("c")

    pltpu.sync_copy(xw_hbm.at[wid], idx_v)

    def zero_body(i, carry):
        for k in range(_DIM // 16):
            accum[i, pl.ds(k * 16, 16)] = jnp.zeros((16,), jnp.float32)
        return carry

    lax.fori_loop(0, _BPW, zero_body, 0)

    def fire(j, carry):
        pltpu.async_copy(table_hbm.at[idx_v.at[j]], accum, sem, add=True)
        return carry

    lax.fori_loop(0, _L, fire, 0)

    def drain(j, carry):
        pltpu.make_async_copy(table_hbm.at[idx_v.at[0]], accum, sem).wait()
        return carry

    lax.fori_loop(0, _L, drain, 0)

    pltpu.sync_copy(accum, out_hbm.at[pl.ds(wid * _BPW, _BPW)])


_sc_pool = functools.partial(
    pl.kernel,
    out_type=jax.ShapeDtypeStruct((_B, _DIM), jnp.float32),
    mesh=plsc.VectorSubcoreMesh(core_axis_name="c", subcore_axis_name="s"),
    scratch_types=[
        pltpu.VMEM((_L, _BPW), jnp.int32),
        pltpu.VMEM((_BPW, _DIM), jnp.float32),
        pltpu.SemaphoreType.DMA,
    ],
    compiler_params=pltpu.CompilerParams(use_tc_tiling_on_sc=False),
)(_sc_pool_body)


def _tc_head_body(ps_ref, x_ref, t0_ref, w_ref, b_ref, o_ref):
    cnt = jnp.sum((x_ref[...] == 0).astype(jnp.float32), axis=1, keepdims=True)
    pooled = ps_ref[...] - cnt * t0_ref[...]
    o_ref[...] = (1.0 / _L) * lax.dot_general(
        pooled, w_ref[...], (((1,), (1,)), ((), ())),
        preferred_element_type=jnp.float32,
    ) + b_ref[...]


_tc_head = pl.pallas_call(
    _tc_head_body,
    out_shape=jax.ShapeDtypeStruct((_B, _NCLS), jnp.float32),
)


def kernel(x, table, W, b):
    xi = x.astype(jnp.int32)
    xw = xi.T.reshape(_L, _NW, _BPW).transpose(1, 0, 2)
    pooled_sum = _sc_pool(table, xw)
    return _tc_head(pooled_sum, xi, table[0:1], W, b.reshape(1, _NCLS))


# speedup vs baseline: 17.3235x; 1.0126x over previous
import functools

import jax
import jax.numpy as jnp
from jax import lax
from jax.experimental import pallas as pl
from jax.experimental.pallas import tpu as pltpu
from jax.experimental.pallas import tpu_sc as plsc

_VOCAB = 100000
_DIM = 64
_NCLS = 100
_B = 4096
_L = 200

_NC = 2
_NS = 16
_NW = _NC * _NS
_BPW = _B // _NW


def _sc_pool_body(table_hbm, x3_hbm, out_hbm, xr_v, idx_v, accum, sem):
    wid = lax.axis_index("s") * _NC + lax.axis_index("c")

    pltpu.sync_copy(x3_hbm.at[wid], xr_v)

    def zero_body(i, carry):
        for k in range(_DIM // 16):
            accum[i, pl.ds(k * 16, 16)] = jnp.zeros((16,), jnp.float32)
        return carry

    lax.fori_loop(0, _BPW, zero_body, 0)

    lanes = lax.iota(jnp.int32, 16)

    def fire(l, carry):
        col = jnp.zeros((16,), jnp.int32) + l
        for i in range(_BPW // 16):
            vals = plsc.load_gather(xr_v, [lanes + (16 * i), col])
            idx_v[l, pl.ds(16 * i, 16)] = vals
        pltpu.async_copy(table_hbm.at[idx_v.at[l]], accum, sem, add=True)
        return carry

    lax.fori_loop(0, _L, fire, 0)

    def drain(j, carry):
        pltpu.make_async_copy(table_hbm.at[idx_v.at[0]], accum, sem).wait()
        return carry

    lax.fori_loop(0, _L, drain, 0)

    pltpu.sync_copy(accum, out_hbm.at[pl.ds(wid * _BPW, _BPW)])


_sc_pool = functools.partial(
    pl.kernel,
    out_type=jax.ShapeDtypeStruct((_B, _DIM), jnp.float32),
    mesh=plsc.VectorSubcoreMesh(core_axis_name="c", subcore_axis_name="s"),
    scratch_types=[
        pltpu.VMEM((_BPW, _L), jnp.int32),
        pltpu.VMEM((_L, _BPW), jnp.int32),
        pltpu.VMEM((_BPW, _DIM), jnp.float32),
        pltpu.SemaphoreType.DMA,
    ],
    compiler_params=pltpu.CompilerParams(
        use_tc_tiling_on_sc=False, needs_layout_passes=False),
)(_sc_pool_body)


def _tc_head_body(ps_ref, x_ref, t0_ref, w_ref, b_ref, o_ref):
    cnt = jnp.sum((x_ref[...] == 0).astype(jnp.float32), axis=1, keepdims=True)
    pooled = ps_ref[...] - cnt * t0_ref[...]
    o_ref[...] = (1.0 / _L) * lax.dot_general(
        pooled, w_ref[...], (((1,), (1,)), ((), ())),
        preferred_element_type=jnp.float32,
    ) + b_ref[...]


_tc_head = pl.pallas_call(
    _tc_head_body,
    out_shape=jax.ShapeDtypeStruct((_B, _NCLS), jnp.float32),
)


def kernel(x, table, W, b):
    xi = x.astype(jnp.int32)
    x3 = xi.reshape(_NW, _BPW, _L)
    pooled_sum = _sc_pool(table, x3)
    return _tc_head(pooled_sum, xi, table[0:1], W, b.reshape(1, _NCLS))


# speedup vs baseline: 18.1681x; 1.0488x over previous
import functools

import jax
import jax.numpy as jnp
from jax import lax
from jax.experimental import pallas as pl
from jax.experimental.pallas import tpu as pltpu
from jax.experimental.pallas import tpu_sc as plsc

_VOCAB = 100000
_DIM = 64
_NCLS = 100
_B = 4096
_L = 200

_NC = 2
_NS = 16
_NW = _NC * _NS
_BPW = _B // _NW


_LP = 256
_DP = 128


def _sc_pool_body(table_hbm, x2_hbm, out_hbm, xr_v, idx_v, accum, sem):
    wid = lax.axis_index("s") * _NC + lax.axis_index("c")

    pltpu.sync_copy(x2_hbm.at[pl.ds(wid * _BPW, _BPW)], xr_v)

    def zero_body(i, carry):
        for k in range(_DIM // 16):
            accum[i, pl.ds(k * 16, 16)] = jnp.zeros((16,), jnp.float32)
        return carry

    lax.fori_loop(0, _BPW, zero_body, 0)

    lanes = lax.iota(jnp.int32, 16)

    def fire(l, carry):
        col = jnp.zeros((16,), jnp.int32) + l
        for i in range(_BPW // 16):
            vals = plsc.load_gather(xr_v, [lanes + (16 * i), col])
            idx_v[l, pl.ds(16 * i, 16)] = vals + vals
        pltpu.async_copy(table_hbm.at[idx_v.at[l]], accum, sem, add=True)
        return carry

    lax.fori_loop(0, _L, fire, 0)

    def drain(j, carry):
        pltpu.make_async_copy(table_hbm.at[idx_v.at[0]], accum, sem).wait()
        return carry

    lax.fori_loop(0, _L, drain, 0)

    pltpu.sync_copy(
        accum, out_hbm.at[pl.ds(wid * _BPW, _BPW), pl.ds(0, _DIM)])


_sc_pool = functools.partial(
    pl.kernel,
    out_type=jax.ShapeDtypeStruct((_B, _DP), jnp.float32),
    mesh=plsc.VectorSubcoreMesh(core_axis_name="c", subcore_axis_name="s"),
    scratch_types=[
        pltpu.VMEM((_BPW, _LP), jnp.int32),
        pltpu.VMEM((_L, _BPW), jnp.int32),
        pltpu.VMEM((_BPW, _DIM), jnp.float32),
        pltpu.SemaphoreType.DMA,
    ],
    name="sc_pool",
    compiler_params=pltpu.CompilerParams(
        use_tc_tiling_on_sc=False, needs_layout_passes=False),
)(_sc_pool_body)


def _tc_head_body(ps_ref, x_ref, t0_ref, w_ref, b_ref, o_ref):
    cnt = jnp.sum((x_ref[...] == 0).astype(jnp.float32), axis=1, keepdims=True)
    pooled = ps_ref[:, : _DIM] - cnt * t0_ref[...]
    o_ref[...] = (1.0 / _L) * lax.dot_general(
        pooled, w_ref[...], (((1,), (1,)), ((), ())),
        preferred_element_type=jnp.float32,
    ) + b_ref[...]


_tc_head = pl.pallas_call(
    _tc_head_body,
    out_shape=jax.ShapeDtypeStruct((_B, _NCLS), jnp.float32),
)


def kernel(x, table, W, b):
    xi = x.astype(jnp.int32)
    xp = jnp.pad(xi, ((0, 0), (0, _LP - _L)))
    tp = jnp.pad(table, ((0, 0), (0, _DP - _DIM))).reshape(2 * _VOCAB, _DIM)
    pooled_sum = _sc_pool(tp, xp)
    return _tc_head(pooled_sum, xi, table[0:1], W, b.reshape(1, _NCLS))


# speedup vs baseline: 18.5215x; 1.0195x over previous
import functools

import jax
import jax.numpy as jnp
from jax import lax
from jax.experimental import pallas as pl
from jax.experimental.pallas import tpu as pltpu
from jax.experimental.pallas import tpu_sc as plsc

_VOCAB = 100000
_DIM = 64
_NCLS = 100
_B = 4096
_L = 200

_NC = 2
_NS = 16
_NW = _NC * _NS
_BPW = _B // _NW


_LP = 256
_DP = 128


def _sc_pool_body(table_hbm, x2_hbm, out_hbm, xr_v, idx_v, accum, sem):
    wid = lax.axis_index("s") * _NC + lax.axis_index("c")

    pltpu.sync_copy(x2_hbm.at[pl.ds(wid * _BPW, _BPW)], xr_v)

    def zero_body(i, carry):
        for k in range(_DIM // 16):
            accum[i, pl.ds(k * 16, 16)] = jnp.zeros((16,), jnp.float32)
        return carry

    lax.fori_loop(0, _BPW, zero_body, 0)

    lanes = lax.iota(jnp.int32, 16)

    def fire(l, carry):
        col = jnp.zeros((16,), jnp.int32) + l
        for i in range(_BPW // 16):
            vals = plsc.load_gather(xr_v, [lanes + (16 * i), col])
            idx_v[l, pl.ds(16 * i, 16)] = vals + vals
        pltpu.async_copy(table_hbm.at[idx_v.at[l]], accum, sem, add=True)
        return carry

    lax.fori_loop(0, _L, fire, 0)

    def drain(j, carry):
        pltpu.make_async_copy(table_hbm.at[idx_v.at[0]], accum, sem).wait()
        return carry

    lax.fori_loop(0, _L, drain, 0)

    pltpu.sync_copy(
        accum, out_hbm.at[pl.ds(wid * _BPW, _BPW), pl.ds(0, _DIM)])


_sc_pool = functools.partial(
    pl.kernel,
    out_type=jax.ShapeDtypeStruct((_B, _DP), jnp.float32),
    mesh=plsc.VectorSubcoreMesh(core_axis_name="c", subcore_axis_name="s"),
    scratch_types=[
        pltpu.VMEM((_BPW, _LP), jnp.int32),
        pltpu.VMEM((_L, _BPW), jnp.int32),
        pltpu.VMEM((_BPW, _DIM), jnp.float32),
        pltpu.SemaphoreType.DMA,
    ],
    name="sc_pool",
    compiler_params=pltpu.CompilerParams(
        use_tc_tiling_on_sc=False, needs_layout_passes=False),
)(_sc_pool_body)


def _tc_head_body(ps_ref, x_ref, t0_ref, w_ref, b_ref, o_ref):
    cnt = jnp.sum((x_ref[...] == 0).astype(jnp.float32), axis=1, keepdims=True)
    pooled = ps_ref[:, : _DIM] - cnt * t0_ref[...]
    o_ref[...] = (1.0 / _L) * lax.dot_general(
        w_ref[...], pooled, (((1,), (1,)), ((), ())),
        preferred_element_type=jnp.float32,
    ) + b_ref[...]


_tc_head = pl.pallas_call(
    _tc_head_body,
    out_shape=jax.ShapeDtypeStruct((_NCLS, _B), jnp.float32),
)


def kernel(x, table, W, b):
    xi = x.astype(jnp.int32)
    xp = jnp.pad(xi, ((0, 0), (0, _LP - _L)))
    tp = jnp.pad(table, ((0, 0), (0, _DP - _DIM))).reshape(2 * _VOCAB, _DIM)
    pooled_sum = _sc_pool(tp, xp)
    logits_t = _tc_head(pooled_sum, xi, table[0:1], W, b.reshape(_NCLS, 1))
    return logits_t.T
